# baseline (speedup 1.0000x reference)
import jax
import jax.numpy as jnp
from jax.experimental import pallas as pl
from jax.experimental.pallas import tpu as pltpu

_BM = 400


def _mm_kernel(mask_ref, oh_ref, out_ref):
    out_ref[...] = jnp.dot(mask_ref[...], oh_ref[...],
                           preferred_element_type=jnp.float32)


def kernel(mask_matrix, x, one_hot_h):
    del x
    n_rows, k = mask_matrix.shape
    n_types = one_hot_h.shape[1]
    return pl.pallas_call(
        _mm_kernel,
        grid=(n_rows // _BM,),
        in_specs=[
            pl.BlockSpec((_BM, k), lambda i: (i, 0)),
            pl.BlockSpec((k, n_types), lambda i: (0, 0)),
        ],
        out_specs=pl.BlockSpec((_BM, n_types), lambda i: (i, 0)),
        out_shape=jax.ShapeDtypeStruct((n_rows, n_types), jnp.float32),
        compiler_params=pltpu.CompilerParams(
            dimension_semantics=("arbitrary",),
        ),
    )(mask_matrix, one_hot_h)
